# baseline (speedup 1.0000x reference)
import functools

import jax
import jax.numpy as jnp
from jax import lax
from jax.experimental import pallas as pl
from jax.experimental.pallas import tpu as pltpu
from jax.experimental.pallas import tpu_sc as plsc

N = 10000
D = 128
E = 320000
NC = 2
NS = 16
K = 128
NW = NC * NS
NBUF = 2
CPT = 80
GRPS = CPT // NBUF
E_PAD = NW * K * CPT
TRASH = N
NACC = 10240
ROWS_PER_TILE = NACC // NS

_mesh = plsc.VectorSubcoreMesh(core_axis_name="c", subcore_axis_name="s")


def _deg_body(dst_hbm, out_hbm, didx, ones_v, zbuf, deg_sh):
    c = lax.axis_index("c")
    s = lax.axis_index("s")
    zv = jnp.zeros((16,), jnp.float32)
    ov = jnp.ones((16,), jnp.float32)
    for j in range(K // 16):
        ones_v[pl.ds(j * 16, 16)] = ov

    def zb(i, carry):
        zbuf[pl.ds(i * 16, 16)] = zv
        return carry

    lax.fori_loop(0, ROWS_PER_TILE // 16, zb, 0)
    pltpu.sync_copy(zbuf, deg_sh.at[pl.ds(s * ROWS_PER_TILE, ROWS_PER_TILE)])
    base = (c * NS + s) * CPT
    pltpu.sync_copy(dst_hbm.at[pl.ds(base, CPT)], didx)
    plsc.subcore_barrier()

    def body(k, carry):
        pltpu.sync_copy(ones_v, deg_sh.at[didx.at[k]], add=True)
        return carry

    lax.fori_loop(0, CPT, body, 0)
    plsc.subcore_barrier()
    pltpu.sync_copy(deg_sh.at[pl.ds(s * ROWS_PER_TILE, ROWS_PER_TILE)],
                    out_hbm.at[c, pl.ds(s * ROWS_PER_TILE, ROWS_PER_TILE)])


_deg_call = functools.partial(
    pl.kernel,
    out_type=jax.ShapeDtypeStruct((NC, NACC), jnp.float32),
    mesh=_mesh,
    scratch_types=[
        pltpu.VMEM((CPT, K), jnp.int32),
        pltpu.VMEM((K,), jnp.float32),
        pltpu.VMEM((ROWS_PER_TILE,), jnp.float32),
        pltpu.VMEM_SHARED((NACC,), jnp.float32),
    ],
)(_deg_body)


def _edge_body(y_hbm, src_hbm, dst_hbm, out_hbm, sidx, didx, rows, acc_sh,
               sem):
    c = lax.axis_index("c")
    s = lax.axis_index("s")
    zv = jnp.zeros((16,), jnp.float32)

    def zero_rows(r, carry):
        for j in range(D // 16):
            rows[r, pl.ds(j * 16, 16)] = zv
        return carry

    lax.fori_loop(0, K, zero_rows, 0)
    for i in range(ROWS_PER_TILE // K):
        pltpu.sync_copy(rows, acc_sh.at[pl.ds(s * ROWS_PER_TILE + i * K, K)])
    plsc.subcore_barrier()

    base = (c * NS + s) * CPT

    def body(k, carry):
        pltpu.sync_copy(src_hbm.at[base + k], sidx)
        pltpu.sync_copy(dst_hbm.at[base + k], didx)
        pltpu.async_copy(y_hbm.at[sidx], rows, sem).wait()
        return carry

    lax.fori_loop(0, CPT, body, 0)
    plsc.subcore_barrier()
    pltpu.sync_copy(acc_sh.at[pl.ds(s * ROWS_PER_TILE, ROWS_PER_TILE)],
                    out_hbm.at[c, pl.ds(s * ROWS_PER_TILE, ROWS_PER_TILE)])


_edge_call = functools.partial(
    pl.kernel,
    out_type=jax.ShapeDtypeStruct((NC, NACC, D), jnp.float32),
    mesh=_mesh,
    scratch_types=[
        pltpu.VMEM((K,), jnp.int32),
        pltpu.VMEM((K,), jnp.int32),
        pltpu.VMEM((K, D), jnp.float32),
        pltpu.VMEM_SHARED((NACC, D), jnp.float32),
        pltpu.SemaphoreType.DMA,
    ],
)(_edge_body)


R = 1000


def _dinv(d0, d1):
    return lax.rsqrt(jnp.maximum(d0 + d1 + 1.0, 1e-12))


def _mm_scale_body(x_ref, w_ref, d0_ref, d1_ref, o_ref):
    d = _dinv(d0_ref[...], d1_ref[...])
    o_ref[...] = jnp.dot(x_ref[...], w_ref[...],
                         preferred_element_type=jnp.float32) * d


def _fuse_body(a0_ref, a1_ref, y1_ref, d0_ref, d1_ref, w_ref, b_ref, o_ref):
    d = _dinv(d0_ref[...], d1_ref[...])
    h = d * (a0_ref[...] + a1_ref[...] + y1_ref[...]) + b_ref[...]
    h = jnp.maximum(h, 0.0)
    o_ref[...] = jnp.dot(h, w_ref[...],
                         preferred_element_type=jnp.float32) * d


def _final_body(a0_ref, a1_ref, y2_ref, d0_ref, d1_ref, b_ref, o_ref):
    d = _dinv(d0_ref[...], d1_ref[...])
    o_ref[...] = d * (a0_ref[...] + a1_ref[...] + y2_ref[...]) + b_ref[...]


_row_spec = pl.BlockSpec((R, D), lambda i: (i, 0))
_deg_spec = pl.BlockSpec((R, 1), lambda i: (i, 0))
_full_spec = pl.BlockSpec((D, D), lambda i: (0, 0))
_bias_spec = pl.BlockSpec((1, D), lambda i: (0, 0))
_out_struct = jax.ShapeDtypeStruct((N, D), jnp.float32)

_mm_scale = pl.pallas_call(
    _mm_scale_body,
    grid=(N // R,),
    in_specs=[_row_spec, _full_spec, _deg_spec, _deg_spec],
    out_specs=_row_spec,
    out_shape=_out_struct,
)

_fuse = pl.pallas_call(
    _fuse_body,
    grid=(N // R,),
    in_specs=[_row_spec, _row_spec, _row_spec, _deg_spec, _deg_spec,
              _full_spec, _bias_spec],
    out_specs=_row_spec,
    out_shape=_out_struct,
)

_final = pl.pallas_call(
    _final_body,
    grid=(N // R,),
    in_specs=[_row_spec, _row_spec, _row_spec, _deg_spec, _deg_spec,
              _bias_spec],
    out_specs=_row_spec,
    out_shape=_out_struct,
)


def kernel(x, edge_index, batch, W1, b1, W2, b2):
    src = edge_index[0].astype(jnp.int32)
    dst = edge_index[1].astype(jnp.int32)
    pad = E_PAD - E
    src_p = jnp.concatenate([src, jnp.zeros((pad,), jnp.int32)])
    dst_p = jnp.concatenate([dst, jnp.full((pad,), TRASH, jnp.int32)])
    src2d = src_p.reshape(E_PAD // K, K)
    dst2d = dst_p.reshape(E_PAD // K, K)

    deg_part = _deg_call(dst2d)
    deg0 = deg_part[0, :N].reshape(N, 1)
    deg1 = deg_part[1, :N].reshape(N, 1)

    y1 = _mm_scale(x, W1, deg0, deg1)
    acc1 = _edge_call(y1, src2d, dst2d)
    y2 = _fuse(acc1[0, :N], acc1[1, :N], y1, deg0, deg1, W2, b1.reshape(1, D))
    acc2 = _edge_call(y2, src2d, dst2d)
    out = _final(acc2[0, :N], acc2[1, :N], y2, deg0, deg1, b2.reshape(1, D))
    return (out, batch)


# speedup vs baseline: 1.1586x; 1.1586x over previous
import functools

import jax
import jax.numpy as jnp
from jax import lax
from jax.experimental import pallas as pl
from jax.experimental.pallas import tpu as pltpu
from jax.experimental.pallas import tpu_sc as plsc

N = 10000
D = 128
E = 320000
NC = 2
NS = 16
K = 128
NW = NC * NS
NBUF = 2
CPT = 80
GRPS = CPT // NBUF
E_PAD = NW * K * CPT
TRASH = N
NACC = 10240
ROWS_PER_TILE = NACC // NS

_mesh = plsc.VectorSubcoreMesh(core_axis_name="c", subcore_axis_name="s")


def _deg_body(dst_hbm, out_hbm, didx, ones_v, zbuf, deg_sh):
    c = lax.axis_index("c")
    s = lax.axis_index("s")
    zv = jnp.zeros((16,), jnp.float32)
    ov = jnp.ones((16,), jnp.float32)
    for j in range(K // 16):
        ones_v[pl.ds(j * 16, 16)] = ov

    def zb(i, carry):
        zbuf[pl.ds(i * 16, 16)] = zv
        return carry

    lax.fori_loop(0, ROWS_PER_TILE // 16, zb, 0)
    pltpu.sync_copy(zbuf, deg_sh.at[pl.ds(s * ROWS_PER_TILE, ROWS_PER_TILE)])
    base = (c * NS + s) * CPT
    pltpu.sync_copy(dst_hbm.at[pl.ds(base, CPT)], didx)
    plsc.subcore_barrier()

    def body(k, carry):
        pltpu.sync_copy(ones_v, deg_sh.at[didx.at[k]], add=True)
        return carry

    lax.fori_loop(0, CPT, body, 0)
    plsc.subcore_barrier()
    pltpu.sync_copy(deg_sh.at[pl.ds(s * ROWS_PER_TILE, ROWS_PER_TILE)],
                    out_hbm.at[c, pl.ds(s * ROWS_PER_TILE, ROWS_PER_TILE)])


_deg_call = functools.partial(
    pl.kernel,
    out_type=jax.ShapeDtypeStruct((NC, NACC), jnp.float32),
    mesh=_mesh,
    scratch_types=[
        pltpu.VMEM((CPT, K), jnp.int32),
        pltpu.VMEM((K,), jnp.float32),
        pltpu.VMEM((ROWS_PER_TILE,), jnp.float32),
        pltpu.VMEM_SHARED((NACC,), jnp.float32),
    ],
)(_deg_body)


def _edge_body(y_hbm, src_hbm, dst_hbm, out_hbm, sbuf, dbuf, rows0, rows1,
               acc_sh, isem0, isem1, gsem0, gsem1):
    rows = (rows0, rows1)
    gsem = (gsem0, gsem1)
    isem = (isem0, isem1)
    c = lax.axis_index("c")
    s = lax.axis_index("s")
    zv = jnp.zeros((16,), jnp.float32)

    def zero_rows(r, carry):
        for j in range(D // 16):
            rows0[r, pl.ds(j * 16, 16)] = zv
        return carry

    lax.fori_loop(0, K, zero_rows, 0)
    for i in range(ROWS_PER_TILE // K):
        pltpu.sync_copy(rows0, acc_sh.at[pl.ds(s * ROWS_PER_TILE + i * K, K)])
    plsc.subcore_barrier()

    base_e = (c * NS + s) * CPT * K

    def prefetch(g, p):
        for b in range(NBUF):
            e0 = base_e + (g * NBUF + b) * K
            pltpu.async_copy(src_hbm.at[pl.ds(e0, K)], sbuf.at[p, b], isem[p])
            pltpu.async_copy(dst_hbm.at[pl.ds(e0, K)], dbuf.at[p, b], isem[p])

    def wait_prefetch(p):
        for _ in range(2 * NBUF):
            pltpu.make_async_copy(src_hbm.at[pl.ds(0, K)], sbuf.at[p, 0],
                                  isem[p]).wait()

    NGRP = CPT // NBUF
    prefetch(0, 0)
    prefetch(1, 1)
    wait_prefetch(0)
    for b in range(NBUF):
        pltpu.async_copy(y_hbm.at[sbuf.at[0, b]], rows[b], gsem[b])

    def outer(gg, carry):
        for p in range(2):
            g = gg * 2 + p
            for b in range(NBUF):
                pltpu.make_async_copy(y_hbm.at[sbuf.at[0, 0]], rows[b],
                                      gsem[b]).wait()
                pltpu.sync_copy(rows[b], acc_sh.at[dbuf.at[p, b]], add=True)

            def do_prefetch(p=p, g=g):
                prefetch(g + 2, p)

            pl.when(g + 2 < NGRP)(do_prefetch)

            def nextgather(p=p, g=g):
                wait_prefetch(1 - p)
                for b in range(NBUF):
                    pltpu.async_copy(y_hbm.at[sbuf.at[1 - p, b]], rows[b],
                                     gsem[b])

            pl.when(g + 1 < NGRP)(nextgather)
        return carry

    lax.fori_loop(0, NGRP // 2, outer, 0)
    plsc.subcore_barrier()
    pltpu.sync_copy(acc_sh.at[pl.ds(s * ROWS_PER_TILE, ROWS_PER_TILE)],
                    out_hbm.at[c, pl.ds(s * ROWS_PER_TILE, ROWS_PER_TILE)])


_edge_call = functools.partial(
    pl.kernel,
    out_type=jax.ShapeDtypeStruct((NC, NACC, D), jnp.float32),
    mesh=_mesh,
    scratch_types=[
        pltpu.VMEM((2, NBUF, K), jnp.int32),
        pltpu.VMEM((2, NBUF, K), jnp.int32),
        pltpu.VMEM((K, D), jnp.float32),
        pltpu.VMEM((K, D), jnp.float32),
        pltpu.VMEM_SHARED((NACC, D), jnp.float32),
        *([pltpu.SemaphoreType.DMA] * 4),
    ],
)(_edge_body)


R = 1000


def _dinv(d0, d1):
    return lax.rsqrt(jnp.maximum(d0 + d1 + 1.0, 1e-12))


def _mm_scale_body(x_ref, w_ref, d0_ref, d1_ref, o_ref):
    d = _dinv(d0_ref[...], d1_ref[...])
    o_ref[...] = jnp.dot(x_ref[...], w_ref[...],
                         preferred_element_type=jnp.float32) * d


def _fuse_body(a0_ref, a1_ref, y1_ref, d0_ref, d1_ref, w_ref, b_ref, o_ref):
    d = _dinv(d0_ref[...], d1_ref[...])
    h = d * (a0_ref[...] + a1_ref[...] + y1_ref[...]) + b_ref[...]
    h = jnp.maximum(h, 0.0)
    o_ref[...] = jnp.dot(h, w_ref[...],
                         preferred_element_type=jnp.float32) * d


def _final_body(a0_ref, a1_ref, y2_ref, d0_ref, d1_ref, b_ref, o_ref):
    d = _dinv(d0_ref[...], d1_ref[...])
    o_ref[...] = d * (a0_ref[...] + a1_ref[...] + y2_ref[...]) + b_ref[...]


_row_spec = pl.BlockSpec((R, D), lambda i: (i, 0))
_deg_spec = pl.BlockSpec((R, 1), lambda i: (i, 0))
_full_spec = pl.BlockSpec((D, D), lambda i: (0, 0))
_bias_spec = pl.BlockSpec((1, D), lambda i: (0, 0))
_out_struct = jax.ShapeDtypeStruct((N, D), jnp.float32)

_mm_scale = pl.pallas_call(
    _mm_scale_body,
    grid=(N // R,),
    in_specs=[_row_spec, _full_spec, _deg_spec, _deg_spec],
    out_specs=_row_spec,
    out_shape=_out_struct,
)

_fuse = pl.pallas_call(
    _fuse_body,
    grid=(N // R,),
    in_specs=[_row_spec, _row_spec, _row_spec, _deg_spec, _deg_spec,
              _full_spec, _bias_spec],
    out_specs=_row_spec,
    out_shape=_out_struct,
)

_final = pl.pallas_call(
    _final_body,
    grid=(N // R,),
    in_specs=[_row_spec, _row_spec, _row_spec, _deg_spec, _deg_spec,
              _bias_spec],
    out_specs=_row_spec,
    out_shape=_out_struct,
)


def kernel(x, edge_index, batch, W1, b1, W2, b2):
    src = edge_index[0].astype(jnp.int32)
    dst = edge_index[1].astype(jnp.int32)
    pad = E_PAD - E
    src_p = jnp.concatenate([src, jnp.zeros((pad,), jnp.int32)])
    dst_p = jnp.concatenate([dst, jnp.full((pad,), TRASH, jnp.int32)])
    dst2d = dst_p.reshape(E_PAD // K, K)

    deg_part = _deg_call(dst2d)
    deg0 = deg_part[0, :N].reshape(N, 1)
    deg1 = deg_part[1, :N].reshape(N, 1)

    y1 = _mm_scale(x, W1, deg0, deg1)
    acc1 = _edge_call(y1, src_p, dst_p)
    y2 = _fuse(acc1[0, :N], acc1[1, :N], y1, deg0, deg1, W2, b1.reshape(1, D))
    acc2 = _edge_call(y2, src_p, dst_p)
    out = _final(acc2[0, :N], acc2[1, :N], y2, deg0, deg1, b2.reshape(1, D))
    return (out, batch)


# speedup vs baseline: 1.2245x; 1.0569x over previous
import functools

import jax
import jax.numpy as jnp
from jax import lax
from jax.experimental import pallas as pl
from jax.experimental.pallas import tpu as pltpu
from jax.experimental.pallas import tpu_sc as plsc

N = 10000
D = 128
E = 320000
NC = 2
NS = 16
K = 128
NW = NC * NS
NBUF = 2
CPT = 80
GRPS = CPT // NBUF
E_PAD = NW * K * CPT
TRASH = N
NACC = 10240
ROWS_PER_TILE = NACC // NS

_mesh = plsc.VectorSubcoreMesh(core_axis_name="c", subcore_axis_name="s")


def _deg_body(dst_hbm, out_hbm, didx, ones_v, zbuf, deg_sh):
    c = lax.axis_index("c")
    s = lax.axis_index("s")
    zv = jnp.zeros((16,), jnp.float32)
    ov = jnp.ones((16,), jnp.float32)
    for j in range(K // 16):
        ones_v[pl.ds(j * 16, 16)] = ov

    def zb(i, carry):
        zbuf[pl.ds(i * 16, 16)] = zv
        return carry

    lax.fori_loop(0, ROWS_PER_TILE // 16, zb, 0)
    pltpu.sync_copy(zbuf, deg_sh.at[pl.ds(s * ROWS_PER_TILE, ROWS_PER_TILE)])
    base = (c * NS + s) * CPT
    pltpu.sync_copy(dst_hbm.at[pl.ds(base, CPT)], didx)
    plsc.subcore_barrier()

    def body(k, carry):
        pltpu.sync_copy(ones_v, deg_sh.at[didx.at[k]], add=True)
        return carry

    lax.fori_loop(0, CPT, body, 0)
    plsc.subcore_barrier()
    pltpu.sync_copy(deg_sh.at[pl.ds(s * ROWS_PER_TILE, ROWS_PER_TILE)],
                    out_hbm.at[c, pl.ds(s * ROWS_PER_TILE, ROWS_PER_TILE)])


_deg_call = functools.partial(
    pl.kernel,
    out_type=jax.ShapeDtypeStruct((NC, NACC), jnp.float32),
    mesh=_mesh,
    scratch_types=[
        pltpu.VMEM((CPT, K), jnp.int32),
        pltpu.VMEM((K,), jnp.float32),
        pltpu.VMEM((ROWS_PER_TILE,), jnp.float32),
        pltpu.VMEM_SHARED((NACC,), jnp.float32),
    ],
)(_deg_body)


def _edge_body(y_hbm, src_hbm, dst_hbm, out_hbm, sbuf, dbuf, rows0, rows1,
               acc_sh, isem0, isem1, gsem0, gsem1):
    rows = (rows0, rows1)
    gsem = (gsem0, gsem1)
    isem = (isem0, isem1)
    c = lax.axis_index("c")
    s = lax.axis_index("s")
    zv = jnp.zeros((16,), jnp.float32)

    def zero_rows(r, carry):
        for j in range(D // 16):
            rows0[r, pl.ds(j * 16, 16)] = zv
        return carry

    lax.fori_loop(0, K, zero_rows, 0)
    for i in range(ROWS_PER_TILE // K):
        pltpu.sync_copy(rows0, acc_sh.at[pl.ds(s * ROWS_PER_TILE + i * K, K)])
    plsc.subcore_barrier()

    base_e = (c * NS + s) * CPT * K

    def prefetch(g, p):
        for b in range(NBUF):
            e0 = base_e + (g * NBUF + b) * K
            pltpu.async_copy(src_hbm.at[pl.ds(e0, K)], sbuf.at[p, b], isem[p])
            pltpu.async_copy(dst_hbm.at[pl.ds(e0, K)], dbuf.at[p, b], isem[p])

    def wait_prefetch(p):
        for _ in range(2 * NBUF):
            pltpu.make_async_copy(src_hbm.at[pl.ds(0, K)], sbuf.at[p, 0],
                                  isem[p]).wait()

    NGRP = CPT // NBUF
    prefetch(0, 0)
    prefetch(1, 1)
    wait_prefetch(0)
    pltpu.async_copy(y_hbm.at[sbuf.at[0, 0]], rows[0], gsem[0])

    def wait_gather(b):
        pltpu.make_async_copy(y_hbm.at[sbuf.at[0, 0]], rows[b],
                              gsem[0]).wait()

    def outer(gg, carry):
        for p in range(2):
            g = gg * 2 + p
            wait_gather(0)
            pltpu.async_copy(y_hbm.at[sbuf.at[p, 1]], rows[1], gsem[0])
            pltpu.sync_copy(rows[0], acc_sh.at[dbuf.at[p, 0]], add=True)
            wait_gather(1)

            def nextgather(p=p):
                wait_prefetch(1 - p)
                pltpu.async_copy(y_hbm.at[sbuf.at[1 - p, 0]], rows[0],
                                 gsem[0])

            pl.when(g + 1 < NGRP)(nextgather)
            pltpu.sync_copy(rows[1], acc_sh.at[dbuf.at[p, 1]], add=True)

            def do_prefetch(p=p, g=g):
                prefetch(g + 2, p)

            pl.when(g + 2 < NGRP)(do_prefetch)
        return carry

    lax.fori_loop(0, NGRP // 2, outer, 0)
    plsc.subcore_barrier()
    pltpu.sync_copy(acc_sh.at[pl.ds(s * ROWS_PER_TILE, ROWS_PER_TILE)],
                    out_hbm.at[c, pl.ds(s * ROWS_PER_TILE, ROWS_PER_TILE)])


_edge_call = functools.partial(
    pl.kernel,
    out_type=jax.ShapeDtypeStruct((NC, NACC, D), jnp.float32),
    mesh=_mesh,
    scratch_types=[
        pltpu.VMEM((2, NBUF, K), jnp.int32),
        pltpu.VMEM((2, NBUF, K), jnp.int32),
        pltpu.VMEM((K, D), jnp.float32),
        pltpu.VMEM((K, D), jnp.float32),
        pltpu.VMEM_SHARED((NACC, D), jnp.float32),
        *([pltpu.SemaphoreType.DMA] * 4),
    ],
)(_edge_body)


R = 1000


def _dinv(d0, d1):
    return lax.rsqrt(jnp.maximum(d0 + d1 + 1.0, 1e-12))


def _mm_scale_body(x_ref, w_ref, d0_ref, d1_ref, o_ref):
    d = _dinv(d0_ref[...], d1_ref[...])
    o_ref[...] = jnp.dot(x_ref[...], w_ref[...],
                         preferred_element_type=jnp.float32) * d


def _fuse_body(a0_ref, a1_ref, y1_ref, d0_ref, d1_ref, w_ref, b_ref, o_ref):
    d = _dinv(d0_ref[...], d1_ref[...])
    h = d * (a0_ref[...] + a1_ref[...] + y1_ref[...]) + b_ref[...]
    h = jnp.maximum(h, 0.0)
    o_ref[...] = jnp.dot(h, w_ref[...],
                         preferred_element_type=jnp.float32) * d


def _final_body(a0_ref, a1_ref, y2_ref, d0_ref, d1_ref, b_ref, o_ref):
    d = _dinv(d0_ref[...], d1_ref[...])
    o_ref[...] = d * (a0_ref[...] + a1_ref[...] + y2_ref[...]) + b_ref[...]


_row_spec = pl.BlockSpec((R, D), lambda i: (i, 0))
_deg_spec = pl.BlockSpec((R, 1), lambda i: (i, 0))
_full_spec = pl.BlockSpec((D, D), lambda i: (0, 0))
_bias_spec = pl.BlockSpec((1, D), lambda i: (0, 0))
_out_struct = jax.ShapeDtypeStruct((N, D), jnp.float32)

_mm_scale = pl.pallas_call(
    _mm_scale_body,
    grid=(N // R,),
    in_specs=[_row_spec, _full_spec, _deg_spec, _deg_spec],
    out_specs=_row_spec,
    out_shape=_out_struct,
)

_fuse = pl.pallas_call(
    _fuse_body,
    grid=(N // R,),
    in_specs=[_row_spec, _row_spec, _row_spec, _deg_spec, _deg_spec,
              _full_spec, _bias_spec],
    out_specs=_row_spec,
    out_shape=_out_struct,
)

_final = pl.pallas_call(
    _final_body,
    grid=(N // R,),
    in_specs=[_row_spec, _row_spec, _row_spec, _deg_spec, _deg_spec,
              _bias_spec],
    out_specs=_row_spec,
    out_shape=_out_struct,
)


def kernel(x, edge_index, batch, W1, b1, W2, b2):
    src = edge_index[0].astype(jnp.int32)
    dst = edge_index[1].astype(jnp.int32)
    pad = E_PAD - E
    src_p = jnp.concatenate([src, jnp.zeros((pad,), jnp.int32)])
    dst_p = jnp.concatenate([dst, jnp.full((pad,), TRASH, jnp.int32)])
    dst2d = dst_p.reshape(E_PAD // K, K)

    deg_part = _deg_call(dst2d)
    deg0 = deg_part[0, :N].reshape(N, 1)
    deg1 = deg_part[1, :N].reshape(N, 1)

    y1 = _mm_scale(x, W1, deg0, deg1)
    acc1 = _edge_call(y1, src_p, dst_p)
    y2 = _fuse(acc1[0, :N], acc1[1, :N], y1, deg0, deg1, W2, b1.reshape(1, D))
    acc2 = _edge_call(y2, src_p, dst_p)
    out = _final(acc2[0, :N], acc2[1, :N], y2, deg0, deg1, b2.reshape(1, D))
    return (out, batch)


# speedup vs baseline: 1.2245x; 1.0000x over previous
import functools

import jax
import jax.numpy as jnp
from jax import lax
from jax.experimental import pallas as pl
from jax.experimental.pallas import tpu as pltpu
from jax.experimental.pallas import tpu_sc as plsc

N = 10000
D = 128
E = 320000
NC = 2
NS = 16
K = 128
NW = NC * NS
NBUF = 2
CPT = 80
GRPS = CPT // NBUF
E_PAD = NW * K * CPT
TRASH = N
NACC = 10240
ROWS_PER_TILE = NACC // NS

_mesh = plsc.VectorSubcoreMesh(core_axis_name="c", subcore_axis_name="s")


def _deg_body(dst_hbm, out_hbm, didx, ones_v, zbuf, deg_sh):
    c = lax.axis_index("c")
    s = lax.axis_index("s")
    zv = jnp.zeros((16,), jnp.float32)
    ov = jnp.ones((16,), jnp.float32)
    for j in range(K // 16):
        ones_v[pl.ds(j * 16, 16)] = ov

    def zb(i, carry):
        zbuf[pl.ds(i * 16, 16)] = zv
        return carry

    lax.fori_loop(0, ROWS_PER_TILE // 16, zb, 0)
    pltpu.sync_copy(zbuf, deg_sh.at[pl.ds(s * ROWS_PER_TILE, ROWS_PER_TILE)])
    base = (c * NS + s) * CPT
    pltpu.sync_copy(dst_hbm.at[pl.ds(base, CPT)], didx)
    plsc.subcore_barrier()

    def body(k, carry):
        pltpu.sync_copy(ones_v, deg_sh.at[didx.at[k]], add=True)
        return carry

    lax.fori_loop(0, CPT, body, 0)
    plsc.subcore_barrier()
    pltpu.sync_copy(deg_sh.at[pl.ds(s * ROWS_PER_TILE, ROWS_PER_TILE)],
                    out_hbm.at[c, pl.ds(s * ROWS_PER_TILE, ROWS_PER_TILE)])


_deg_call = functools.partial(
    pl.kernel,
    out_type=jax.ShapeDtypeStruct((NC, NACC), jnp.float32),
    mesh=_mesh,
    scratch_types=[
        pltpu.VMEM((CPT, K), jnp.int32),
        pltpu.VMEM((K,), jnp.float32),
        pltpu.VMEM((ROWS_PER_TILE,), jnp.float32),
        pltpu.VMEM_SHARED((NACC,), jnp.float32),
    ],
)(_deg_body)


def _edge_body(y_hbm, src_hbm, dst_hbm, out_hbm, s00, s01, s10, s11,
               d00, d01, d10, d11, rows0, rows1, acc_sh, isem0, isem1, gsem):
    sbuf = ((s00, s01), (s10, s11))
    dbuf = ((d00, d01), (d10, d11))
    rows = (rows0, rows1)
    isem = (isem0, isem1)
    c = lax.axis_index("c")
    s = lax.axis_index("s")
    zv = jnp.zeros((16,), jnp.float32)

    def zero_rows(r, carry):
        for j in range(D // 16):
            rows0[r, pl.ds(j * 16, 16)] = zv
        return carry

    lax.fori_loop(0, K, zero_rows, 0)
    for i in range(ROWS_PER_TILE // K):
        pltpu.sync_copy(rows0, acc_sh.at[pl.ds(s * ROWS_PER_TILE + i * K, K)])
    plsc.subcore_barrier()

    base_e = (c * NS + s) * CPT * K

    def prefetch(g, p):
        for b in range(NBUF):
            e0 = base_e + (g * NBUF + b) * K
            pltpu.async_copy(src_hbm.at[pl.ds(e0, K)], sbuf[p][b], isem[p])
            pltpu.async_copy(dst_hbm.at[pl.ds(e0, K)], dbuf[p][b], isem[p])

    def wait_prefetch(p):
        for _ in range(2 * NBUF):
            pltpu.make_async_copy(src_hbm.at[pl.ds(0, K)], sbuf[p][0],
                                  isem[p]).wait()

    NGRP = CPT // NBUF
    prefetch(0, 0)
    prefetch(1, 1)
    wait_prefetch(0)
    pltpu.async_copy(y_hbm.at[sbuf[0][0]], rows[0], gsem)

    def wait_gather(b):
        pltpu.make_async_copy(y_hbm.at[sbuf[0][0]], rows[b], gsem).wait()

    def outer(gg, carry):
        for p in range(2):
            g = gg * 2 + p
            wait_gather(0)
            pltpu.async_copy(y_hbm.at[sbuf[p][1]], rows[1], gsem)
            pltpu.sync_copy(rows[0], acc_sh.at[dbuf[p][0]], add=True)
            wait_gather(1)

            def nextgather(p=p):
                wait_prefetch(1 - p)
                pltpu.async_copy(y_hbm.at[sbuf[1 - p][0]], rows[0], gsem)

            pl.when(g + 1 < NGRP)(nextgather)
            pltpu.sync_copy(rows[1], acc_sh.at[dbuf[p][1]], add=True)

            def do_prefetch(p=p, g=g):
                prefetch(g + 2, p)

            pl.when(g + 2 < NGRP)(do_prefetch)
        return carry

    lax.fori_loop(0, NGRP // 2, outer, 0)
    plsc.subcore_barrier()
    pltpu.sync_copy(acc_sh.at[pl.ds(s * ROWS_PER_TILE, ROWS_PER_TILE)],
                    out_hbm.at[c, pl.ds(s * ROWS_PER_TILE, ROWS_PER_TILE)])


_edge_call = functools.partial(
    pl.kernel,
    out_type=jax.ShapeDtypeStruct((NC, NACC, D), jnp.float32),
    mesh=_mesh,
    scratch_types=[
        *([pltpu.VMEM((K,), jnp.int32)] * 8),
        pltpu.VMEM((K, D), jnp.float32),
        pltpu.VMEM((K, D), jnp.float32),
        pltpu.VMEM_SHARED((NACC, D), jnp.float32),
        *([pltpu.SemaphoreType.DMA] * 3),
    ],
)(_edge_body)


R = 1000


def _dinv(d0, d1):
    return lax.rsqrt(jnp.maximum(d0 + d1 + 1.0, 1e-12))


def _mm_scale_body(x_ref, w_ref, d0_ref, d1_ref, o_ref):
    d = _dinv(d0_ref[...], d1_ref[...])
    o_ref[...] = jnp.dot(x_ref[...], w_ref[...],
                         preferred_element_type=jnp.float32) * d


def _fuse_body(a0_ref, a1_ref, y1_ref, d0_ref, d1_ref, w_ref, b_ref, o_ref):
    d = _dinv(d0_ref[...], d1_ref[...])
    h = d * (a0_ref[...] + a1_ref[...] + y1_ref[...]) + b_ref[...]
    h = jnp.maximum(h, 0.0)
    o_ref[...] = jnp.dot(h, w_ref[...],
                         preferred_element_type=jnp.float32) * d


def _final_body(a0_ref, a1_ref, y2_ref, d0_ref, d1_ref, b_ref, o_ref):
    d = _dinv(d0_ref[...], d1_ref[...])
    o_ref[...] = d * (a0_ref[...] + a1_ref[...] + y2_ref[...]) + b_ref[...]


_row_spec = pl.BlockSpec((R, D), lambda i: (i, 0))
_deg_spec = pl.BlockSpec((R, 1), lambda i: (i, 0))
_full_spec = pl.BlockSpec((D, D), lambda i: (0, 0))
_bias_spec = pl.BlockSpec((1, D), lambda i: (0, 0))
_out_struct = jax.ShapeDtypeStruct((N, D), jnp.float32)

_mm_scale = pl.pallas_call(
    _mm_scale_body,
    grid=(N // R,),
    in_specs=[_row_spec, _full_spec, _deg_spec, _deg_spec],
    out_specs=_row_spec,
    out_shape=_out_struct,
)

_fuse = pl.pallas_call(
    _fuse_body,
    grid=(N // R,),
    in_specs=[_row_spec, _row_spec, _row_spec, _deg_spec, _deg_spec,
              _full_spec, _bias_spec],
    out_specs=_row_spec,
    out_shape=_out_struct,
)

_final = pl.pallas_call(
    _final_body,
    grid=(N // R,),
    in_specs=[_row_spec, _row_spec, _row_spec, _deg_spec, _deg_spec,
              _bias_spec],
    out_specs=_row_spec,
    out_shape=_out_struct,
)


def kernel(x, edge_index, batch, W1, b1, W2, b2):
    src = edge_index[0].astype(jnp.int32)
    dst = edge_index[1].astype(jnp.int32)
    pad = E_PAD - E
    src_p = jnp.concatenate([src, jnp.zeros((pad,), jnp.int32)])
    dst_p = jnp.concatenate([dst, jnp.full((pad,), TRASH, jnp.int32)])
    dst2d = dst_p.reshape(E_PAD // K, K)

    deg_part = _deg_call(dst2d)
    deg0 = deg_part[0, :N].reshape(N, 1)
    deg1 = deg_part[1, :N].reshape(N, 1)

    y1 = _mm_scale(x, W1, deg0, deg1)
    acc1 = _edge_call(y1, src_p, dst_p)
    y2 = _fuse(acc1[0, :N], acc1[1, :N], y1, deg0, deg1, W2, b1.reshape(1, D))
    acc2 = _edge_call(y2, src_p, dst_p)
    out = _final(acc2[0, :N], acc2[1, :N], y2, deg0, deg1, b2.reshape(1, D))
    return (out, batch)
